# baseline (speedup 1.0000x reference)
import jax
import jax.numpy as jnp
from jax import lax
from jax.experimental import pallas as pl
from jax.experimental.pallas import tpu as pltpu
from jax.experimental.pallas import tpu_sc as plsc

_N = 10000
_NP = 10240
_FW = 3 * _NP
_E = 640000
_NC = 2
_NS = 16
_L = 16
_NW = _NC * _NS
_EPW = _E // _NW
_CH = 2000
_NCHUNK = _EPW // _CH
_NPAIR = _NCHUNK // 2
_STEPS = _CH // _L
_SL = _FW // _NS
_CHA = 2176


def _body(pos_h, ei_h, eps_h, sig_h,
          outf_h,
          pos_v, facc, sd_v, eps_v, sig_v,
          acc_v, tmp_v, shared_all, sem0, sem1, sem2):
    c = lax.axis_index("c")
    s = lax.axis_index("s")
    wid = s * _NC + c
    base = wid * _EPW

    def aligned(off):
        aoff = (off // 128) * 128
        return jnp.minimum(aoff, _E - _CHA)

    def issue(off, b, sem):
        half = pl.ds(b * _CH, _CH)
        pltpu.async_copy(ei_h.at[:, pl.ds(aligned(off), _CHA)],
                         sd_v.at[:, pl.ds(b * _CHA, _CHA)], sem)
        pltpu.async_copy(eps_h.at[pl.ds(off, _CH)], eps_v.at[half], sem)
        pltpu.async_copy(sig_h.at[pl.ds(off, _CH)], sig_v.at[half], sem)

    def drain(b, sem):
        z = pl.ds(0, _CH)
        half = pl.ds(b * _CH, _CH)
        pltpu.make_async_copy(ei_h.at[:, pl.ds(0, _CHA)],
                              sd_v.at[:, pl.ds(b * _CHA, _CHA)], sem).wait()
        pltpu.make_async_copy(eps_h.at[z], eps_v.at[half], sem).wait()
        pltpu.make_async_copy(sig_h.at[z], sig_v.at[half], sem).wait()

    issue(base, 0, sem0)
    issue(base + _CH, 1, sem1)

    pcp = [pltpu.async_copy(pos_h, pos_v, sem2)]

    zero_f = jnp.zeros((_L,), jnp.float32)

    @plsc.parallel_loop(0, _FW // _L, unroll=8)
    def zfill(j):
        facc[pl.ds(j * _L, _L)] = zero_f

    for cp in pcp:
        cp.wait()

    off1 = jnp.full((_L,), _NP, jnp.int32)
    off2 = jnp.full((_L,), 2 * _NP, jnp.int32)
    rc1 = jnp.full((_L,), 1, jnp.int32)
    rc2 = jnp.full((_L,), 2, jnp.int32)

    def edge_block(b, off, e_acc):
        cbase = b * _CH
        ibase = b * _CHA + (off - aligned(off))

        @plsc.parallel_loop(0, _STEPS, unroll=5, carry=e_acc)
        def step(i, e_c):
            sl = pl.ds(cbase + i * _L, _L)
            sli = pl.ds(ibase + i * _L, _L)
            si = sd_v[0, sli]
            di = sd_v[1, sli]
            ep = eps_v[sl]
            sg = sig_v[sl]
            s3 = si * 3
            d3 = di * 3
            dx = plsc.load_gather(pos_v, [s3]) - plsc.load_gather(pos_v, [d3])
            dy = plsc.load_gather(pos_v, [s3 + rc1]) - plsc.load_gather(pos_v, [d3 + rc1])
            dz = plsc.load_gather(pos_v, [s3 + rc2]) - plsc.load_gather(pos_v, [d3 + rc2])
            r2 = dx * dx + dy * dy + dz * dz
            inv = 1.0 / jnp.maximum(r2, 0.09)
            s2 = sg * sg * inv
            s6 = s2 * s2 * s2
            s12 = s6 * s6
            e4 = 4.0 * ep
            g = e4 * (12.0 * s12 - 6.0 * s6) * inv
            plsc.addupdate_scatter(facc, [si], g * dx)
            plsc.addupdate_scatter(facc, [si + off1], g * dy)
            plsc.addupdate_scatter(facc, [si + off2], g * dz)
            return e_c + e4 * (s12 - s6)
        return step

    def pair(j, e_acc):
        offj = base + (2 * j) * _CH
        drain(0, sem0)
        e_acc = edge_block(0, offj, e_acc)

        @pl.when(j < _NPAIR - 1)
        def _():
            issue(offj + 2 * _CH, 0, sem0)

        drain(1, sem1)
        e_acc = edge_block(1, offj + _CH, e_acc)

        @pl.when(j < _NPAIR - 1)
        def _():
            issue(offj + 3 * _CH, 1, sem1)

        return e_acc

    e_total = lax.fori_loop(0, _NPAIR, pair, jnp.zeros((_L,), jnp.float32))

    erow = jnp.where(wid < 30, wid // 15, 2)
    ecol = jnp.where(wid < 30, wid % 15, wid - 30)
    facc[pl.ds(erow * _NP + _N + ecol * _L, _L)] = e_total

    pltpu.sync_copy(facc, shared_all.at[pl.ds(s * _FW, _FW)])
    plsc.subcore_barrier()

    sbase = s * _SL
    pltpu.sync_copy(shared_all.at[pl.ds(sbase, _SL)], acc_v)

    def red(t, carry):
        pltpu.sync_copy(shared_all.at[pl.ds(t * _FW + sbase, _SL)], tmp_v)

        @plsc.parallel_loop(0, _SL // _L, unroll=8)
        def add_vec(j):
            jl = pl.ds(j * _L, _L)
            acc_v[jl] = acc_v[jl] + tmp_v[jl]
        return carry

    lax.fori_loop(1, _NS, red, 0)
    pltpu.sync_copy(acc_v, outf_h.at[c, pl.ds(sbase, _SL)])


@jax.jit
def _lj(pos, ei, eps, sig):
    mesh = plsc.VectorSubcoreMesh(core_axis_name="c", subcore_axis_name="s")
    f = pl.kernel(
        _body,
        out_type=jax.ShapeDtypeStruct((_NC, _FW), jnp.float32),
        mesh=mesh,
        scratch_types=[
            pltpu.VMEM((3 * _N,), jnp.float32),
            pltpu.VMEM((_FW,), jnp.float32),
            pltpu.VMEM((2, 2 * _CHA), jnp.int32),
            pltpu.VMEM((2 * _CH,), jnp.float32),
            pltpu.VMEM((2 * _CH,), jnp.float32),
            pltpu.VMEM((_SL,), jnp.float32),
            pltpu.VMEM((_SL,), jnp.float32),
            pltpu.VMEM_SHARED((_NS * _FW,), jnp.float32),
            pltpu.SemaphoreType.DMA,
            pltpu.SemaphoreType.DMA,
            pltpu.SemaphoreType.DMA,
        ],
        compiler_params=pltpu.CompilerParams(needs_layout_passes=False),
    )
    return f(pos, ei, eps, sig)


def kernel(pos, epsilon, sigma, edge_index):
    outf = _lj(pos.reshape(-1), edge_index, epsilon, sigma)
    ftot = (outf[0] + outf[1]).reshape(3, _NP)
    forces = ftot[:, :_N].T
    energy = ftot[:, _N:].sum()
    return energy, forces


# speedup vs baseline: 1.0704x; 1.0704x over previous
import jax
import jax.numpy as jnp
from jax import lax
from jax.experimental import pallas as pl
from jax.experimental.pallas import tpu as pltpu
from jax.experimental.pallas import tpu_sc as plsc

_N = 10000
_NP = 10240
_FW = 3 * _NP
_E = 640000
_NC = 2
_NS = 16
_L = 16
_NW = _NC * _NS
_EPW = _E // _NW
_CH = 2000
_NCHUNK = _EPW // _CH
_NPAIR = _NCHUNK // 2
_STEPS = _CH // _L
_SL = _FW // _NS
_CHA = 2176


def _body(posx_h, posy_h, posz_h, ei_h, eps_h, sig_h,
          outf_h,
          x_v, y_v, z_v, facc, sd_v, eps_v, sig_v,
          acc_v, tmp_v, shared_all, sem0, sem1, sem2):
    c = lax.axis_index("c")
    s = lax.axis_index("s")
    wid = s * _NC + c
    base = wid * _EPW

    def aligned(off):
        aoff = (off // 128) * 128
        return jnp.minimum(aoff, _E - _CHA)

    def issue(off, b, sem):
        half = pl.ds(b * _CH, _CH)
        pltpu.async_copy(ei_h.at[:, pl.ds(aligned(off), _CHA)],
                         sd_v.at[:, pl.ds(b * _CHA, _CHA)], sem)
        pltpu.async_copy(eps_h.at[pl.ds(off, _CH)], eps_v.at[half], sem)
        pltpu.async_copy(sig_h.at[pl.ds(off, _CH)], sig_v.at[half], sem)

    def drain(b, sem):
        z = pl.ds(0, _CH)
        half = pl.ds(b * _CH, _CH)
        pltpu.make_async_copy(ei_h.at[:, pl.ds(0, _CHA)],
                              sd_v.at[:, pl.ds(b * _CHA, _CHA)], sem).wait()
        pltpu.make_async_copy(eps_h.at[z], eps_v.at[half], sem).wait()
        pltpu.make_async_copy(sig_h.at[z], sig_v.at[half], sem).wait()

    issue(base, 0, sem0)
    issue(base + _CH, 1, sem1)

    pcp = [pltpu.async_copy(posx_h, x_v, sem2),
           pltpu.async_copy(posy_h, y_v, sem2),
           pltpu.async_copy(posz_h, z_v, sem2)]

    zero_f = jnp.zeros((_L,), jnp.float32)

    @plsc.parallel_loop(0, _FW // _L, unroll=8)
    def zfill(j):
        facc[pl.ds(j * _L, _L)] = zero_f

    for cp in pcp:
        cp.wait()

    off1 = jnp.full((_L,), _NP, jnp.int32)
    off2 = jnp.full((_L,), 2 * _NP, jnp.int32)

    def edge_block(b, off):
        cbase = b * _CH
        ibase = b * _CHA + (off - aligned(off))

        @plsc.parallel_loop(0, _STEPS, step=5)
        def step(i):
            for k in range(5):
                sl = pl.ds(cbase + (i + k) * _L, _L)
                sli = pl.ds(ibase + (i + k) * _L, _L)
                si = sd_v[0, sli]
                di = sd_v[1, sli]
                ep = eps_v[sl]
                sg = sig_v[sl]
                dx = plsc.load_gather(x_v, [si]) - plsc.load_gather(x_v, [di])
                dy = plsc.load_gather(y_v, [si]) - plsc.load_gather(y_v, [di])
                dz = plsc.load_gather(z_v, [si]) - plsc.load_gather(z_v, [di])
                r2 = dx * dx + dy * dy + dz * dz
                inv = 1.0 / jnp.maximum(r2, 0.09)
                s2 = sg * sg * inv
                s6 = s2 * s2 * s2
                s12 = s6 * s6
                e4 = 4.0 * ep
                g = e4 * (12.0 * s12 - 6.0 * s6) * inv
                plsc.addupdate_scatter(facc, [si], g * dx)
                plsc.addupdate_scatter(facc, [si + off1], g * dy)
                plsc.addupdate_scatter(facc, [si + off2], g * dz)
                plsc.addupdate(facc.at[pl.ds(_N + k * _L, _L)],
                               e4 * (s12 - s6))

    def pair(j, carry):
        offj = base + (2 * j) * _CH
        drain(0, sem0)
        edge_block(0, offj)

        @pl.when(j < _NPAIR - 1)
        def _():
            issue(offj + 2 * _CH, 0, sem0)

        drain(1, sem1)
        edge_block(1, offj + _CH)

        @pl.when(j < _NPAIR - 1)
        def _():
            issue(offj + 3 * _CH, 1, sem1)

        return carry

    lax.fori_loop(0, _NPAIR, pair, 0)

    pltpu.sync_copy(facc, shared_all.at[pl.ds(s * _FW, _FW)])
    plsc.subcore_barrier()

    sbase = s * _SL
    pltpu.sync_copy(shared_all.at[pl.ds(sbase, _SL)], acc_v)

    def red(t, carry):
        pltpu.sync_copy(shared_all.at[pl.ds(t * _FW + sbase, _SL)], tmp_v)

        @plsc.parallel_loop(0, _SL // _L, unroll=8)
        def add_vec(j):
            jl = pl.ds(j * _L, _L)
            acc_v[jl] = acc_v[jl] + tmp_v[jl]
        return carry

    lax.fori_loop(1, _NS, red, 0)
    pltpu.sync_copy(acc_v, outf_h.at[c, pl.ds(sbase, _SL)])


@jax.jit
def _lj(posx, posy, posz, ei, eps, sig):
    mesh = plsc.VectorSubcoreMesh(core_axis_name="c", subcore_axis_name="s")
    f = pl.kernel(
        _body,
        out_type=jax.ShapeDtypeStruct((_NC, _FW), jnp.float32),
        mesh=mesh,
        scratch_types=[
            pltpu.VMEM((_N,), jnp.float32),
            pltpu.VMEM((_N,), jnp.float32),
            pltpu.VMEM((_N,), jnp.float32),
            pltpu.VMEM((_FW,), jnp.float32),
            pltpu.VMEM((2, 2 * _CHA), jnp.int32),
            pltpu.VMEM((2 * _CH,), jnp.float32),
            pltpu.VMEM((2 * _CH,), jnp.float32),
            pltpu.VMEM((_SL,), jnp.float32),
            pltpu.VMEM((_SL,), jnp.float32),
            pltpu.VMEM_SHARED((_NS * _FW,), jnp.float32),
            pltpu.SemaphoreType.DMA,
            pltpu.SemaphoreType.DMA,
            pltpu.SemaphoreType.DMA,
        ],
        compiler_params=pltpu.CompilerParams(needs_layout_passes=False),
    )
    return f(posx, posy, posz, ei, eps, sig)


def kernel(pos, epsilon, sigma, edge_index):
    outf = _lj(pos[:, 0], pos[:, 1], pos[:, 2],
               edge_index, epsilon, sigma)
    ftot = (outf[0] + outf[1]).reshape(3, _NP)
    forces = ftot[:, :_N].T
    energy = ftot[:, _N:].sum()
    return energy, forces


# speedup vs baseline: 1.1612x; 1.0849x over previous
import jax
import jax.numpy as jnp
from jax import lax
from jax.experimental import pallas as pl
from jax.experimental.pallas import tpu as pltpu
from jax.experimental.pallas import tpu_sc as plsc

_N = 10000
_NP = 10240
_FW = 3 * _NP
_E = 640000
_NC = 2
_NS = 16
_L = 16
_NW = _NC * _NS
_EPW = _E // _NW
_CH = 2000
_NCHUNK = _EPW // _CH
_NPAIR = _NCHUNK // 2
_STEPS = _CH // _L
_SL = _FW // _NS
_CHA = 2176


def _body(posx_h, posy_h, posz_h, ei_h, eps_h, sig_h,
          outf_h,
          x_v, y_v, z_v, facc, sd_v, eps_v, sig_v,
          acc_v, tmp_v, shared_all, sem0, sem1, sem2):
    c = lax.axis_index("c")
    s = lax.axis_index("s")
    wid = s * _NC + c
    base = wid * _EPW

    def aligned(off):
        aoff = (off // 128) * 128
        return jnp.minimum(aoff, _E - _CHA)

    def issue(off, b, sem):
        half = pl.ds(b * _CH, _CH)
        pltpu.async_copy(ei_h.at[:, pl.ds(aligned(off), _CHA)],
                         sd_v.at[:, pl.ds(b * _CHA, _CHA)], sem)
        pltpu.async_copy(eps_h.at[pl.ds(off, _CH)], eps_v.at[half], sem)
        pltpu.async_copy(sig_h.at[pl.ds(off, _CH)], sig_v.at[half], sem)

    def drain(b, sem):
        z = pl.ds(0, _CH)
        half = pl.ds(b * _CH, _CH)
        pltpu.make_async_copy(ei_h.at[:, pl.ds(0, _CHA)],
                              sd_v.at[:, pl.ds(b * _CHA, _CHA)], sem).wait()
        pltpu.make_async_copy(eps_h.at[z], eps_v.at[half], sem).wait()
        pltpu.make_async_copy(sig_h.at[z], sig_v.at[half], sem).wait()

    issue(base, 0, sem0)
    issue(base + _CH, 1, sem1)

    pcp = [pltpu.async_copy(posx_h, x_v, sem2),
           pltpu.async_copy(posy_h, y_v, sem2),
           pltpu.async_copy(posz_h, z_v, sem2)]

    zero_f = jnp.zeros((_L,), jnp.float32)

    @plsc.parallel_loop(0, _FW // _L, unroll=8)
    def zfill(j):
        facc[pl.ds(j * _L, _L)] = zero_f

    for cp in pcp:
        cp.wait()

    off1 = jnp.full((_L,), _NP, jnp.int32)
    off2 = jnp.full((_L,), 2 * _NP, jnp.int32)

    def edge_block(b, off, e_acc):
        cbase = b * _CH
        ibase = b * _CHA + (off - aligned(off))

        @plsc.parallel_loop(0, _STEPS, unroll=25, carry=e_acc)
        def step(i, e_c):
            sl = pl.ds(cbase + i * _L, _L)
            sli = pl.ds(ibase + i * _L, _L)
            si = sd_v[0, sli]
            di = sd_v[1, sli]
            ep = eps_v[sl]
            sg = sig_v[sl]
            dx = plsc.load_gather(x_v, [si]) - plsc.load_gather(x_v, [di])
            dy = plsc.load_gather(y_v, [si]) - plsc.load_gather(y_v, [di])
            dz = plsc.load_gather(z_v, [si]) - plsc.load_gather(z_v, [di])
            r2 = dx * dx + dy * dy + dz * dz
            inv = 1.0 / jnp.maximum(r2, 0.09)
            s2 = sg * sg * inv
            s6 = s2 * s2 * s2
            s12 = s6 * s6
            e4 = 4.0 * ep
            g = e4 * (12.0 * s12 - 6.0 * s6) * inv
            plsc.addupdate_scatter(facc, [si], g * dx)
            plsc.addupdate_scatter(facc, [si + off1], g * dy)
            plsc.addupdate_scatter(facc, [si + off2], g * dz)
            return e_c + e4 * (s12 - s6)
        return step

    def pair(j, e_acc):
        offj = base + (2 * j) * _CH
        drain(0, sem0)
        e_acc = edge_block(0, offj, e_acc)

        @pl.when(j < _NPAIR - 1)
        def _():
            issue(offj + 2 * _CH, 0, sem0)

        drain(1, sem1)
        e_acc = edge_block(1, offj + _CH, e_acc)

        @pl.when(j < _NPAIR - 1)
        def _():
            issue(offj + 3 * _CH, 1, sem1)

        return e_acc

    e_total = lax.fori_loop(0, _NPAIR, pair, jnp.zeros((_L,), jnp.float32))

    erow = jnp.where(wid < 30, wid // 15, 2)
    ecol = jnp.where(wid < 30, wid % 15, wid - 30)
    facc[pl.ds(erow * _NP + _N + ecol * _L, _L)] = e_total

    pltpu.sync_copy(facc, shared_all.at[pl.ds(s * _FW, _FW)])
    plsc.subcore_barrier()

    sbase = s * _SL
    pltpu.sync_copy(shared_all.at[pl.ds(sbase, _SL)], acc_v)

    def red(t, carry):
        pltpu.sync_copy(shared_all.at[pl.ds(t * _FW + sbase, _SL)], tmp_v)

        @plsc.parallel_loop(0, _SL // _L, unroll=8)
        def add_vec(j):
            jl = pl.ds(j * _L, _L)
            acc_v[jl] = acc_v[jl] + tmp_v[jl]
        return carry

    lax.fori_loop(1, _NS, red, 0)
    pltpu.sync_copy(acc_v, outf_h.at[c, pl.ds(sbase, _SL)])


@jax.jit
def _lj(posx, posy, posz, ei, eps, sig):
    mesh = plsc.VectorSubcoreMesh(core_axis_name="c", subcore_axis_name="s")
    f = pl.kernel(
        _body,
        out_type=jax.ShapeDtypeStruct((_NC, _FW), jnp.float32),
        mesh=mesh,
        scratch_types=[
            pltpu.VMEM((_N,), jnp.float32),
            pltpu.VMEM((_N,), jnp.float32),
            pltpu.VMEM((_N,), jnp.float32),
            pltpu.VMEM((_FW,), jnp.float32),
            pltpu.VMEM((2, 2 * _CHA), jnp.int32),
            pltpu.VMEM((2 * _CH,), jnp.float32),
            pltpu.VMEM((2 * _CH,), jnp.float32),
            pltpu.VMEM((_SL,), jnp.float32),
            pltpu.VMEM((_SL,), jnp.float32),
            pltpu.VMEM_SHARED((_NS * _FW,), jnp.float32),
            pltpu.SemaphoreType.DMA,
            pltpu.SemaphoreType.DMA,
            pltpu.SemaphoreType.DMA,
        ],
        compiler_params=pltpu.CompilerParams(needs_layout_passes=False),
    )
    return f(posx, posy, posz, ei, eps, sig)


def kernel(pos, epsilon, sigma, edge_index):
    outf = _lj(pos[:, 0], pos[:, 1], pos[:, 2],
               edge_index, epsilon, sigma)
    ftot = (outf[0] + outf[1]).reshape(3, _NP)
    forces = ftot[:, :_N].T
    energy = ftot[:, _N:].sum()
    return energy, forces


# speedup vs baseline: 1.1839x; 1.0196x over previous
import jax
import jax.numpy as jnp
from jax import lax
from jax.experimental import pallas as pl
from jax.experimental.pallas import tpu as pltpu
from jax.experimental.pallas import tpu_sc as plsc

_N = 10000
_NP = 10240
_FW = 3 * _NP
_E = 640000
_NC = 2
_NS = 16
_L = 16
_NW = _NC * _NS
_EPW = _E // _NW
_CH = 2000
_NCHUNK = _EPW // _CH
_NPAIR = _NCHUNK // 2
_STEPS = _CH // _L
_SL = _FW // _NS
_CHA = 2176


def _body(posx_h, posy_h, posz_h, ei_h, eps_h, sig_h,
          outf_h,
          x_v, y_v, z_v, facc, sd_v, eps_v, sig_v,
          acc_v, tmp_v, shared_all, sem0, sem1, sem2):
    c = lax.axis_index("c")
    s = lax.axis_index("s")
    wid = s * _NC + c
    base = wid * _EPW

    def aligned(off):
        aoff = (off // 128) * 128
        return jnp.minimum(aoff, _E - _CHA)

    def issue(off, b, sem):
        half = pl.ds(b * _CH, _CH)
        pltpu.async_copy(ei_h.at[:, pl.ds(aligned(off), _CHA)],
                         sd_v.at[:, pl.ds(b * _CHA, _CHA)], sem)
        pltpu.async_copy(eps_h.at[pl.ds(off, _CH)], eps_v.at[half], sem)
        pltpu.async_copy(sig_h.at[pl.ds(off, _CH)], sig_v.at[half], sem)

    def drain(b, sem):
        z = pl.ds(0, _CH)
        half = pl.ds(b * _CH, _CH)
        pltpu.make_async_copy(ei_h.at[:, pl.ds(0, _CHA)],
                              sd_v.at[:, pl.ds(b * _CHA, _CHA)], sem).wait()
        pltpu.make_async_copy(eps_h.at[z], eps_v.at[half], sem).wait()
        pltpu.make_async_copy(sig_h.at[z], sig_v.at[half], sem).wait()

    issue(base, 0, sem0)
    issue(base + _CH, 1, sem1)

    pcp = [pltpu.async_copy(posx_h, x_v, sem2),
           pltpu.async_copy(posy_h, y_v, sem2),
           pltpu.async_copy(posz_h, z_v, sem2)]

    zero_f = jnp.zeros((_L,), jnp.float32)

    @plsc.parallel_loop(0, _FW // _L, unroll=8)
    def zfill(j):
        facc[pl.ds(j * _L, _L)] = zero_f

    for cp in pcp:
        cp.wait()

    off1 = jnp.full((_L,), _NP, jnp.int32)
    off2 = jnp.full((_L,), 2 * _NP, jnp.int32)

    def edge_block(b, off, e_acc):
        cbase = b * _CH
        ibase = b * _CHA + (off - aligned(off))

        @plsc.parallel_loop(0, _STEPS, unroll=5, carry=e_acc)
        def step(i, e_c):
            sl = pl.ds(cbase + i * _L, _L)
            sli = pl.ds(ibase + i * _L, _L)
            si = sd_v[0, sli]
            di = sd_v[1, sli]
            ep = eps_v[sl]
            sg = sig_v[sl]
            dx = plsc.load_gather(x_v, [si]) - plsc.load_gather(x_v, [di])
            dy = plsc.load_gather(y_v, [si]) - plsc.load_gather(y_v, [di])
            dz = plsc.load_gather(z_v, [si]) - plsc.load_gather(z_v, [di])
            r2 = dx * dx + dy * dy + dz * dz
            inv = 1.0 / jnp.maximum(r2, 0.09)
            s2 = sg * sg * inv
            s6 = s2 * s2 * s2
            s12 = s6 * s6
            e4 = 4.0 * ep
            g = e4 * (12.0 * s12 - 6.0 * s6) * inv
            plsc.addupdate_scatter(facc, [si], g * dx)
            plsc.addupdate_scatter(facc, [si + off1], g * dy)
            plsc.addupdate_scatter(facc, [si + off2], g * dz)
            return e_c + e4 * (s12 - s6)
        return step

    def pair(j, e_acc):
        offj = base + (2 * j) * _CH
        drain(0, sem0)
        e_acc = edge_block(0, offj, e_acc)

        @pl.when(j < _NPAIR - 1)
        def _():
            issue(offj + 2 * _CH, 0, sem0)

        drain(1, sem1)
        e_acc = edge_block(1, offj + _CH, e_acc)

        @pl.when(j < _NPAIR - 1)
        def _():
            issue(offj + 3 * _CH, 1, sem1)

        return e_acc

    e_total = lax.fori_loop(0, _NPAIR, pair, jnp.zeros((_L,), jnp.float32))

    erow = jnp.where(wid < 30, wid // 15, 2)
    ecol = jnp.where(wid < 30, wid % 15, wid - 30)
    facc[pl.ds(erow * _NP + _N + ecol * _L, _L)] = e_total

    pltpu.sync_copy(facc, shared_all.at[pl.ds(s * _FW, _FW)])
    plsc.subcore_barrier()

    sbase = s * _SL
    pltpu.sync_copy(shared_all.at[pl.ds(sbase, _SL)], acc_v)

    def red(t, carry):
        pltpu.sync_copy(shared_all.at[pl.ds(t * _FW + sbase, _SL)], tmp_v)

        @plsc.parallel_loop(0, _SL // _L, unroll=8)
        def add_vec(j):
            jl = pl.ds(j * _L, _L)
            acc_v[jl] = acc_v[jl] + tmp_v[jl]
        return carry

    lax.fori_loop(1, _NS, red, 0)
    pltpu.sync_copy(acc_v, outf_h.at[c, pl.ds(sbase, _SL)])


@jax.jit
def _lj(posx, posy, posz, ei, eps, sig):
    mesh = plsc.VectorSubcoreMesh(core_axis_name="c", subcore_axis_name="s")
    f = pl.kernel(
        _body,
        out_type=jax.ShapeDtypeStruct((_NC, _FW), jnp.float32),
        mesh=mesh,
        scratch_types=[
            pltpu.VMEM((_N,), jnp.float32),
            pltpu.VMEM((_N,), jnp.float32),
            pltpu.VMEM((_N,), jnp.float32),
            pltpu.VMEM((_FW,), jnp.float32),
            pltpu.VMEM((2, 2 * _CHA), jnp.int32),
            pltpu.VMEM((2 * _CH,), jnp.float32),
            pltpu.VMEM((2 * _CH,), jnp.float32),
            pltpu.VMEM((_SL,), jnp.float32),
            pltpu.VMEM((_SL,), jnp.float32),
            pltpu.VMEM_SHARED((_NS * _FW,), jnp.float32),
            pltpu.SemaphoreType.DMA,
            pltpu.SemaphoreType.DMA,
            pltpu.SemaphoreType.DMA,
        ],
        compiler_params=pltpu.CompilerParams(needs_layout_passes=False),
    )
    return f(posx, posy, posz, ei, eps, sig)


def kernel(pos, epsilon, sigma, edge_index):
    outf = _lj(pos[:, 0], pos[:, 1], pos[:, 2],
               edge_index, epsilon, sigma)
    ftot = (outf[0] + outf[1]).reshape(3, _NP)
    forces = ftot[:, :_N].T
    energy = ftot[:, _N:].sum()
    return energy, forces
